# baseline (speedup 1.0000x reference)
import jax
import jax.numpy as jnp
from jax.experimental import pallas as pl
from jax.experimental.pallas import tpu as pltpu

_TBLK = 1568
_KBLK = 512


def _vq_body(z_ref, cbt_ref, cb_ref, zq_ref, idx_ref):
    z = z_ref[...]
    t, d = z.shape
    kdim = cbt_ref.shape[1]
    nchunks = kdim // _KBLK
    hi = jax.lax.Precision.HIGHEST
    inf = jnp.float32(jnp.inf)

    cbt = cbt_ref[...]
    cn_full = jnp.sum(cbt * cbt, axis=0, keepdims=True)

    m1 = i1 = m2 = i2 = None
    for c in range(nchunks):
        lo_k, hi_k = c * _KBLK, (c + 1) * _KBLK
        ctc = cbt[:, lo_k:hi_k]
        cn = cn_full[:, lo_k:hi_k]
        dot = jax.lax.dot_general(
            z, ctc, (((1,), (0,)), ((), ())),
            preferred_element_type=jnp.float32, precision=hi)
        s = cn - 2.0 * dot
        kio = jax.lax.broadcasted_iota(jnp.int32, s.shape, 1) + lo_k
        a1 = jnp.min(s, axis=1, keepdims=True)
        j1 = jnp.min(jnp.where(s == a1, kio, kdim), axis=1, keepdims=True)
        s2 = jnp.where(kio == j1, inf, s)
        a2 = jnp.min(s2, axis=1, keepdims=True)
        j2 = jnp.min(jnp.where(s2 == a2, kio, kdim), axis=1, keepdims=True)
        if c == 0:
            m1, i1, m2, i2 = a1, j1, a2, j2
        else:
            cond = a1 < m1
            n_m2 = jnp.where(cond, jnp.where(a2 < m1, a2, m1),
                             jnp.where(a1 < m2, a1, m2))
            n_i2 = jnp.where(cond, jnp.where(a2 < m1, j2, i1),
                             jnp.where(a1 < m2, j1, i2))
            m1 = jnp.where(cond, a1, m1)
            i1 = jnp.where(cond, j1, i1)
            m2, i2 = n_m2, n_i2

    def exact_rows(idx):
        acc = None
        for c in range(nchunks):
            lo_k = c * _KBLK
            kio = jax.lax.broadcasted_iota(jnp.int32, (t, _KBLK), 1) + lo_k
            oh = (kio == idx).astype(jnp.float32)
            g = jax.lax.dot_general(
                oh, cb_ref[pl.ds(lo_k, _KBLK), :],
                (((1,), (0,)), ((), ())),
                preferred_element_type=jnp.float32, precision=hi)
            acc = g if acc is None else acc + g
        return acc

    c1 = exact_rows(i1)
    c2 = exact_rows(i2)

    d1 = jnp.sum((z - c1) ** 2, axis=1, keepdims=True)
    d2 = jnp.sum((z - c2) ** 2, axis=1, keepdims=True)
    pick2 = (d2 < d1) | ((d2 == d1) & (i2 < i1))
    zq = jnp.where(pick2, c2, c1)
    idx_ref[...] = jnp.where(pick2, i2, i1)
    zq_ref[...] = z + (zq - z)


def kernel(z_e, codebook):
    b, t, d = z_e.shape
    k = codebook.shape[0]
    n = b * t
    z2 = z_e.reshape(n, d)
    cbt = codebook.T
    zq2, idx2 = pl.pallas_call(
        _vq_body,
        grid=(n // _TBLK,),
        in_specs=[
            pl.BlockSpec((_TBLK, d), lambda i: (i, 0)),
            pl.BlockSpec((d, k), lambda i: (0, 0)),
            pl.BlockSpec((k, d), lambda i: (0, 0)),
        ],
        out_specs=[
            pl.BlockSpec((_TBLK, d), lambda i: (i, 0)),
            pl.BlockSpec((_TBLK, 1), lambda i: (i, 0)),
        ],
        out_shape=[
            jax.ShapeDtypeStruct((n, d), jnp.float32),
            jax.ShapeDtypeStruct((n, 1), jnp.int32),
        ],
        compiler_params=pltpu.CompilerParams(
            dimension_semantics=("parallel",)),
    )(z2, cbt, codebook)
    return zq2.reshape(b, t, d), idx2.reshape(b, t)


# speedup vs baseline: 1.0271x; 1.0271x over previous
import jax
import jax.numpy as jnp
from jax.experimental import pallas as pl
from jax.experimental.pallas import tpu as pltpu

_TBLK = 784
_KBLK = 512


def _vq_body(z_ref, cb_ref, zq_ref, idx_ref):
    z = z_ref[...]
    t, d = z.shape
    kdim = cb_ref.shape[0]
    nchunks = kdim // _KBLK
    hi = jax.lax.Precision.HIGHEST
    inf = jnp.float32(jnp.inf)

    cbt = jnp.transpose(cb_ref[...])
    cn_full = jnp.sum(cbt * cbt, axis=0, keepdims=True)

    m1 = i1 = m2 = i2 = None
    for c in range(nchunks):
        lo_k, hi_k = c * _KBLK, (c + 1) * _KBLK
        ctc = cbt[:, lo_k:hi_k]
        cn = cn_full[:, lo_k:hi_k]
        dot = jax.lax.dot_general(
            z, ctc, (((1,), (0,)), ((), ())),
            preferred_element_type=jnp.float32, precision=hi)
        s = cn - 2.0 * dot
        kio = jax.lax.broadcasted_iota(jnp.int32, s.shape, 1) + lo_k
        a1 = jnp.min(s, axis=1, keepdims=True)
        j1 = jnp.min(jnp.where(s == a1, kio, kdim), axis=1, keepdims=True)
        s2 = jnp.where(kio == j1, inf, s)
        a2 = jnp.min(s2, axis=1, keepdims=True)
        j2 = jnp.min(jnp.where(s2 == a2, kio, kdim), axis=1, keepdims=True)
        if c == 0:
            m1, i1, m2, i2 = a1, j1, a2, j2
        else:
            cond = a1 < m1
            n_m2 = jnp.where(cond, jnp.where(a2 < m1, a2, m1),
                             jnp.where(a1 < m2, a1, m2))
            n_i2 = jnp.where(cond, jnp.where(a2 < m1, j2, i1),
                             jnp.where(a1 < m2, j1, i2))
            m1 = jnp.where(cond, a1, m1)
            i1 = jnp.where(cond, j1, i1)
            m2, i2 = n_m2, n_i2

    def exact_rows(idx):
        acc = None
        for c in range(nchunks):
            lo_k = c * _KBLK
            kio = jax.lax.broadcasted_iota(jnp.int32, (t, _KBLK), 1) + lo_k
            oh = (kio == idx).astype(jnp.float32)
            g = jax.lax.dot_general(
                oh, cb_ref[pl.ds(lo_k, _KBLK), :],
                (((1,), (0,)), ((), ())),
                preferred_element_type=jnp.float32, precision=hi)
            acc = g if acc is None else acc + g
        return acc

    c1 = exact_rows(i1)
    c2 = exact_rows(i2)

    d1 = jnp.sum((z - c1) ** 2, axis=1, keepdims=True)
    d2 = jnp.sum((z - c2) ** 2, axis=1, keepdims=True)
    pick2 = (d2 < d1) | ((d2 == d1) & (i2 < i1))
    zq = jnp.where(pick2, c2, c1)
    idx_ref[...] = jnp.where(pick2, i2, i1)
    zq_ref[...] = z + (zq - z)


def kernel(z_e, codebook):
    b, t, d = z_e.shape
    k = codebook.shape[0]
    n = b * t
    z2 = z_e.reshape(n, d)
    zq2, idx2 = pl.pallas_call(
        _vq_body,
        grid=(n // _TBLK,),
        in_specs=[
            pl.BlockSpec((_TBLK, d), lambda i: (i, 0)),
            pl.BlockSpec((k, d), lambda i: (0, 0)),
        ],
        out_specs=[
            pl.BlockSpec((_TBLK, d), lambda i: (i, 0)),
            pl.BlockSpec((_TBLK, 1), lambda i: (i, 0)),
        ],
        out_shape=[
            jax.ShapeDtypeStruct((n, d), jnp.float32),
            jax.ShapeDtypeStruct((n, 1), jnp.int32),
        ],
        compiler_params=pltpu.CompilerParams(
            dimension_semantics=("parallel",)),
    )(z2, codebook)
    return zq2.reshape(b, t, d), idx2.reshape(b, t)
